# baseline (speedup 1.0000x reference)
import functools

import jax
import jax.numpy as jnp
from jax import lax
from jax.experimental import pallas as pl
from jax.experimental.pallas import tpu as pltpu
from jax.experimental.pallas import tpu_sc as plsc


def _repack_body(x_ref, o_ref):
    x3 = x_ref[...].reshape(32, 256, 4)
    o_ref[...] = jnp.concatenate([x3[:, :, q].T for q in range(4)], axis=1)


def _repack(tableT):
    V = tableT.shape[1]
    nb = (V + 1023) // 1024
    return pl.pallas_call(
        _repack_body,
        grid=(nb,),
        in_specs=[pl.BlockSpec((32, 1024), lambda i: (0, i))],
        out_specs=pl.BlockSpec((256, 128), lambda i: (i, 0)),
        out_shape=jax.ShapeDtypeStruct((nb * 256, 128), jnp.float32),
    )(tableT)


def _sc_gather_packed(table4, idx):
    B = idx.shape[0]
    D4 = table4.shape[1]
    info = plsc.get_sparse_core_info()
    NC, NS = info.num_cores, info.num_subcores
    L = info.num_lanes
    NW = NC * NS
    b_per_w = B // NW
    mesh = plsc.VectorSubcoreMesh(core_axis_name="c", subcore_axis_name="s")

    @functools.partial(
        pl.kernel,
        mesh=mesh,
        out_type=jax.ShapeDtypeStruct((B, D4), jnp.float32),
        scratch_types=[
            pltpu.VMEM((b_per_w,), jnp.int32),
            pltpu.VMEM((b_per_w,), jnp.int32),
            pltpu.VMEM((b_per_w, D4), jnp.float32),
            pltpu.SemaphoreType.DMA,
        ],
    )
    def gather_kernel(idx_hbm, table_hbm, out_hbm, idx_v, idx4_v, rows_v, sem):
        wid = lax.axis_index("s") * NC + lax.axis_index("c")
        base = wid * b_per_w
        pltpu.sync_copy(idx_hbm.at[pl.ds(base, b_per_w)], idx_v)
        for i in range(b_per_w // L):
            idx4_v[pl.ds(i * L, L)] = idx_v[pl.ds(i * L, L)] >> 2
        pltpu.async_copy(table_hbm.at[idx4_v], rows_v, sem).wait()
        pltpu.sync_copy(rows_v, out_hbm.at[pl.ds(base, b_per_w)])

    return gather_kernel(idx, table4)


_BT = 2048


def _mlp_body(p_ref, uid_ref, d_ref, h_ref, m_ref, dtab_ref, htab_ref,
              w1u_ref, w1d_ref, w1h_ref, w1m_ref, b1_ref, w2_ref, b2_ref,
              w3_ref, b3_ref, o_ref):
    f32 = jnp.float32
    bt = p_ref.shape[0]
    ncat = dtab_ref.shape[0]
    D = dtab_ref.shape[1]
    sub = uid_ref[...] & 3
    packed = p_ref[...]
    uemb = jnp.where(sub == 0, packed[:, :D], 0.0)
    for k in range(1, 4):
        uemb = uemb + jnp.where(sub == k, packed[:, k * D:(k + 1) * D], 0.0)
    doh = (d_ref[...] == lax.broadcasted_iota(jnp.int32, (bt, ncat), 1)).astype(f32)
    hoh = (h_ref[...] == lax.broadcasted_iota(jnp.int32, (bt, ncat), 1)).astype(f32)
    demb = jnp.dot(doh, dtab_ref[...], preferred_element_type=f32)
    hemb = jnp.dot(hoh, htab_ref[...], preferred_element_type=f32)
    acc = jnp.dot(uemb, w1u_ref[...], preferred_element_type=f32)
    acc = acc + jnp.dot(demb, w1d_ref[...], preferred_element_type=f32)
    acc = acc + jnp.dot(hemb, w1h_ref[...], preferred_element_type=f32)
    acc = acc + m_ref[...] * w1m_ref[...]
    h1 = jnp.maximum(acc + b1_ref[...], 0.0)
    h2 = jnp.maximum(
        jnp.dot(h1, w2_ref[...], preferred_element_type=f32) + b2_ref[...], 0.0)
    o_ref[...] = jnp.dot(h2, w3_ref[...], preferred_element_type=f32) + b3_ref[...]


def _mlp_call(packed, uid2, days2, hours2, md2, dtab, htab, w1u, w1d, w1h,
              w1m, b1r, w2, b2r, w3, b3r):
    B = packed.shape[0]
    n_out = w3.shape[1]
    bt = _BT
    grid = (B // bt,)

    def row_block(cols):
        return pl.BlockSpec((bt, cols), lambda i: (i, 0))

    def full(a):
        return pl.BlockSpec(a.shape, lambda i: (0,) * a.ndim)

    return pl.pallas_call(
        _mlp_body,
        grid=grid,
        in_specs=[
            row_block(packed.shape[1]),
            row_block(1), row_block(1), row_block(1), row_block(1),
            full(dtab), full(htab), full(w1u), full(w1d), full(w1h),
            full(w1m), full(b1r), full(w2), full(b2r), full(w3), full(b3r),
        ],
        out_specs=row_block(n_out),
        out_shape=jax.ShapeDtypeStruct((B, n_out), jnp.float32),
    )(packed, uid2, days2, hours2, md2, dtab, htab, w1u, w1d, w1h, w1m, b1r,
      w2, b2r, w3, b3r)


def kernel(user_ids, hours, days, move_distance, user_table, day_table,
           hour_table, W1, b1, W2, b2, W3, b3):
    B = user_ids.shape[0]
    D = user_table.shape[1]
    f32 = jnp.float32

    uid32 = user_ids.astype(jnp.int32)
    table4 = _repack(user_table.T)
    packed = _sc_gather_packed(table4, uid32)

    ncat = 32
    dtab = jnp.zeros((ncat, D), f32).at[: day_table.shape[0]].set(day_table)
    htab = jnp.zeros((ncat, D), f32).at[: hour_table.shape[0]].set(hour_table)

    w1u = W1[:, :D].T
    w1d = W1[:, D:2 * D].T
    w1h = W1[:, 2 * D:3 * D].T
    w1m = W1[:, 3 * D][None, :]
    b1r = b1[None, :]
    w2 = W2.T
    b2r = b2[None, :]
    n_out = 8
    w3 = jnp.zeros((W2.shape[0], n_out), f32).at[:, : W3.shape[0]].set(W3.T)
    b3r = jnp.zeros((1, n_out), f32).at[0, : W3.shape[0]].set(b3)

    uid2 = uid32[:, None]
    days2 = days.astype(jnp.int32)[:, None]
    hours2 = hours.astype(jnp.int32)[:, None]
    md2 = move_distance[:, None]

    out = _mlp_call(packed, uid2, days2, hours2, md2, dtab, htab, w1u, w1d,
                    w1h, w1m, b1r, w2, b2r, w3, b3r)
    return out[:, : W3.shape[0]]


# speedup vs baseline: 17.2604x; 17.2604x over previous
import functools

import jax
import jax.numpy as jnp
from jax import lax
from jax.experimental import pallas as pl
from jax.experimental.pallas import tpu as pltpu
from jax.experimental.pallas import tpu_sc as plsc


_RW = 8192


def _repack_body(x_ref, o_ref):
    f32 = jnp.float32
    eye = (lax.broadcasted_iota(jnp.int32, (32, 32), 0)
           == lax.broadcasted_iota(jnp.int32, (32, 32), 1)).astype(f32)
    x3 = x_ref[...].reshape(32, 4, _RW // 4)
    parts = [
        lax.dot_general(x3[:, q, :], eye, (((0,), (0,)), ((), ())),
                        preferred_element_type=f32)
        for q in range(4)
    ]
    o_ref[...] = jnp.concatenate(parts, axis=1)


def _repack(tableT):
    V = tableT.shape[1]
    nb = (V + _RW - 1) // _RW
    return pl.pallas_call(
        _repack_body,
        grid=(nb,),
        in_specs=[pl.BlockSpec((32, _RW), lambda i: (0, i))],
        out_specs=pl.BlockSpec((_RW // 4, 128), lambda i: (i, 0)),
        out_shape=jax.ShapeDtypeStruct((nb * (_RW // 4), 128), jnp.float32),
    )(tableT)


def _sc_gather_packed(table4, idx):
    B = idx.shape[0]
    D4 = table4.shape[1]
    info = plsc.get_sparse_core_info()
    NC, NS = info.num_cores, info.num_subcores
    L = info.num_lanes
    NW = NC * NS
    b_per_w = B // NW
    mesh = plsc.VectorSubcoreMesh(core_axis_name="c", subcore_axis_name="s")

    @functools.partial(
        pl.kernel,
        mesh=mesh,
        out_type=jax.ShapeDtypeStruct((B, D4), jnp.float32),
        scratch_types=[
            pltpu.VMEM((b_per_w,), jnp.int32),
            pltpu.VMEM((b_per_w,), jnp.int32),
            pltpu.VMEM((b_per_w, D4), jnp.float32),
            pltpu.SemaphoreType.DMA,
        ],
    )
    def gather_kernel(idx_hbm, table_hbm, out_hbm, idx_v, idx4_v, rows_v, sem):
        wid = lax.axis_index("s") * NC + lax.axis_index("c")
        base = wid * b_per_w
        pltpu.sync_copy(idx_hbm.at[pl.ds(base, b_per_w)], idx_v)
        for i in range(b_per_w // L):
            v = idx_v[pl.ds(i * L, L)]
            idx4_v[pl.ds(i * L, L)] = ((v >> 13) << 11) + (v & 2047)
        pltpu.async_copy(table_hbm.at[idx4_v], rows_v, sem).wait()
        pltpu.sync_copy(rows_v, out_hbm.at[pl.ds(base, b_per_w)])

    return gather_kernel(idx, table4)


_BT = 2048


def _mlp_body(p_ref, uid_ref, d_ref, h_ref, m_ref, dtab_ref, htab_ref,
              w1u_ref, w1d_ref, w1h_ref, w1m_ref, b1_ref, w2_ref, b2_ref,
              w3_ref, b3_ref, o_ref):
    f32 = jnp.float32
    bt = p_ref.shape[0]
    ncat = dtab_ref.shape[0]
    D = dtab_ref.shape[1]
    sub = (uid_ref[...] >> 11) & 3
    packed = p_ref[...]
    uemb = jnp.where(sub == 0, packed[:, :D], 0.0)
    for k in range(1, 4):
        uemb = uemb + jnp.where(sub == k, packed[:, k * D:(k + 1) * D], 0.0)
    doh = (d_ref[...] == lax.broadcasted_iota(jnp.int32, (bt, ncat), 1)).astype(f32)
    hoh = (h_ref[...] == lax.broadcasted_iota(jnp.int32, (bt, ncat), 1)).astype(f32)
    demb = jnp.dot(doh, dtab_ref[...], preferred_element_type=f32)
    hemb = jnp.dot(hoh, htab_ref[...], preferred_element_type=f32)
    acc = jnp.dot(uemb, w1u_ref[...], preferred_element_type=f32)
    acc = acc + jnp.dot(demb, w1d_ref[...], preferred_element_type=f32)
    acc = acc + jnp.dot(hemb, w1h_ref[...], preferred_element_type=f32)
    acc = acc + m_ref[...] * w1m_ref[...]
    h1 = jnp.maximum(acc + b1_ref[...], 0.0)
    h2 = jnp.maximum(
        jnp.dot(h1, w2_ref[...], preferred_element_type=f32) + b2_ref[...], 0.0)
    o_ref[...] = jnp.dot(h2, w3_ref[...], preferred_element_type=f32) + b3_ref[...]


def _mlp_call(packed, uid2, days2, hours2, md2, dtab, htab, w1u, w1d, w1h,
              w1m, b1r, w2, b2r, w3, b3r):
    B = packed.shape[0]
    n_out = w3.shape[1]
    bt = _BT
    grid = (B // bt,)

    def row_block(cols):
        return pl.BlockSpec((bt, cols), lambda i: (i, 0))

    def full(a):
        return pl.BlockSpec(a.shape, lambda i: (0,) * a.ndim)

    return pl.pallas_call(
        _mlp_body,
        grid=grid,
        in_specs=[
            row_block(packed.shape[1]),
            row_block(1), row_block(1), row_block(1), row_block(1),
            full(dtab), full(htab), full(w1u), full(w1d), full(w1h),
            full(w1m), full(b1r), full(w2), full(b2r), full(w3), full(b3r),
        ],
        out_specs=row_block(n_out),
        out_shape=jax.ShapeDtypeStruct((B, n_out), jnp.float32),
    )(packed, uid2, days2, hours2, md2, dtab, htab, w1u, w1d, w1h, w1m, b1r,
      w2, b2r, w3, b3r)


def kernel(user_ids, hours, days, move_distance, user_table, day_table,
           hour_table, W1, b1, W2, b2, W3, b3):
    B = user_ids.shape[0]
    D = user_table.shape[1]
    f32 = jnp.float32

    uid32 = user_ids.astype(jnp.int32)
    table4 = _repack(user_table.T)
    packed = _sc_gather_packed(table4, uid32)

    ncat = 32
    dtab = jnp.zeros((ncat, D), f32).at[: day_table.shape[0]].set(day_table)
    htab = jnp.zeros((ncat, D), f32).at[: hour_table.shape[0]].set(hour_table)

    w1u = W1[:, :D].T
    w1d = W1[:, D:2 * D].T
    w1h = W1[:, 2 * D:3 * D].T
    w1m = W1[:, 3 * D][None, :]
    b1r = b1[None, :]
    w2 = W2.T
    b2r = b2[None, :]
    n_out = 8
    w3 = jnp.zeros((W2.shape[0], n_out), f32).at[:, : W3.shape[0]].set(W3.T)
    b3r = jnp.zeros((1, n_out), f32).at[0, : W3.shape[0]].set(b3)

    uid2 = uid32[:, None]
    days2 = days.astype(jnp.int32)[:, None]
    hours2 = hours.astype(jnp.int32)[:, None]
    md2 = move_distance[:, None]

    out = _mlp_call(packed, uid2, days2, hours2, md2, dtab, htab, w1u, w1d,
                    w1h, w1m, b1r, w2, b2r, w3, b3r)
    return out[:, : W3.shape[0]]


# speedup vs baseline: 17.9691x; 1.0411x over previous
import functools

import jax
import jax.numpy as jnp
from jax import lax
from jax.experimental import pallas as pl
from jax.experimental.pallas import tpu as pltpu
from jax.experimental.pallas import tpu_sc as plsc


_RW = 16384
_SH_BLK = _RW.bit_length() - 1
_SH_Q = _SH_BLK - 2
_MQ = (1 << _SH_Q) - 1


def _repack_body(x_ref, o_ref):
    f32 = jnp.float32
    eye = (lax.broadcasted_iota(jnp.int32, (32, 32), 0)
           == lax.broadcasted_iota(jnp.int32, (32, 32), 1)).astype(f32)
    w = _RW // 4
    for q in range(4):
        o_ref[:, 32 * q:32 * (q + 1)] = lax.dot_general(
            x_ref[:, q * w:(q + 1) * w], eye, (((0,), (0,)), ((), ())),
            preferred_element_type=f32)


def _repack(tableT):
    V = tableT.shape[1]
    nb = (V + _RW - 1) // _RW
    return pl.pallas_call(
        _repack_body,
        grid=(nb,),
        in_specs=[pl.BlockSpec((32, _RW), lambda i: (0, i))],
        out_specs=pl.BlockSpec((_RW // 4, 128), lambda i: (i, 0)),
        out_shape=jax.ShapeDtypeStruct((nb * (_RW // 4), 128), jnp.float32),
        compiler_params=pltpu.CompilerParams(
            fuse_transposed_lhs_in_matmul=True),
    )(tableT)


def _sc_gather_packed(table4, idx):
    B = idx.shape[0]
    D4 = table4.shape[1]
    info = plsc.get_sparse_core_info()
    NC, NS = info.num_cores, info.num_subcores
    L = info.num_lanes
    NW = NC * NS
    b_per_w = B // NW
    mesh = plsc.VectorSubcoreMesh(core_axis_name="c", subcore_axis_name="s")

    @functools.partial(
        pl.kernel,
        mesh=mesh,
        out_type=jax.ShapeDtypeStruct((B, D4), jnp.float32),
        scratch_types=[
            pltpu.VMEM((b_per_w,), jnp.int32),
            pltpu.VMEM((b_per_w,), jnp.int32),
            pltpu.VMEM((b_per_w, D4), jnp.float32),
            pltpu.SemaphoreType.DMA,
        ],
    )
    def gather_kernel(idx_hbm, table_hbm, out_hbm, idx_v, idx4_v, rows_v, sem):
        wid = lax.axis_index("s") * NC + lax.axis_index("c")
        base = wid * b_per_w
        pltpu.sync_copy(idx_hbm.at[pl.ds(base, b_per_w)], idx_v)
        for i in range(b_per_w // L):
            v = idx_v[pl.ds(i * L, L)]
            idx4_v[pl.ds(i * L, L)] = ((v >> _SH_BLK) << _SH_Q) + (v & _MQ)
        pltpu.async_copy(table_hbm.at[idx4_v], rows_v, sem).wait()
        pltpu.sync_copy(rows_v, out_hbm.at[pl.ds(base, b_per_w)])

    return gather_kernel(idx, table4)


_BT = 2048


def _mlp_body(p_ref, uid_ref, d_ref, h_ref, m_ref, dtab_ref, htab_ref,
              w1u_ref, w1d_ref, w1h_ref, w1m_ref, b1_ref, w2_ref, b2_ref,
              w3_ref, b3_ref, o_ref):
    f32 = jnp.float32
    bt = p_ref.shape[0]
    ncat = dtab_ref.shape[0]
    D = dtab_ref.shape[1]
    sub = (uid_ref[...] >> _SH_Q) & 3
    packed = p_ref[...]
    uemb = jnp.where(sub == 0, packed[:, :D], 0.0)
    for k in range(1, 4):
        uemb = uemb + jnp.where(sub == k, packed[:, k * D:(k + 1) * D], 0.0)
    doh = (d_ref[...] == lax.broadcasted_iota(jnp.int32, (bt, ncat), 1)).astype(f32)
    hoh = (h_ref[...] == lax.broadcasted_iota(jnp.int32, (bt, ncat), 1)).astype(f32)
    demb = jnp.dot(doh, dtab_ref[...], preferred_element_type=f32)
    hemb = jnp.dot(hoh, htab_ref[...], preferred_element_type=f32)
    acc = jnp.dot(uemb, w1u_ref[...], preferred_element_type=f32)
    acc = acc + jnp.dot(demb, w1d_ref[...], preferred_element_type=f32)
    acc = acc + jnp.dot(hemb, w1h_ref[...], preferred_element_type=f32)
    acc = acc + m_ref[...] * w1m_ref[...]
    h1 = jnp.maximum(acc + b1_ref[...], 0.0)
    h2 = jnp.maximum(
        jnp.dot(h1, w2_ref[...], preferred_element_type=f32) + b2_ref[...], 0.0)
    o_ref[...] = jnp.dot(h2, w3_ref[...], preferred_element_type=f32) + b3_ref[...]


def _mlp_call(packed, uid2, days2, hours2, md2, dtab, htab, w1u, w1d, w1h,
              w1m, b1r, w2, b2r, w3, b3r):
    B = packed.shape[0]
    n_out = w3.shape[1]
    bt = _BT
    grid = (B // bt,)

    def row_block(cols):
        return pl.BlockSpec((bt, cols), lambda i: (i, 0))

    def full(a):
        return pl.BlockSpec(a.shape, lambda i: (0,) * a.ndim)

    return pl.pallas_call(
        _mlp_body,
        grid=grid,
        in_specs=[
            row_block(packed.shape[1]),
            row_block(1), row_block(1), row_block(1), row_block(1),
            full(dtab), full(htab), full(w1u), full(w1d), full(w1h),
            full(w1m), full(b1r), full(w2), full(b2r), full(w3), full(b3r),
        ],
        out_specs=row_block(n_out),
        out_shape=jax.ShapeDtypeStruct((B, n_out), jnp.float32),
    )(packed, uid2, days2, hours2, md2, dtab, htab, w1u, w1d, w1h, w1m, b1r,
      w2, b2r, w3, b3r)


def kernel(user_ids, hours, days, move_distance, user_table, day_table,
           hour_table, W1, b1, W2, b2, W3, b3):
    B = user_ids.shape[0]
    D = user_table.shape[1]
    f32 = jnp.float32

    uid32 = user_ids.astype(jnp.int32)
    table4 = _repack(user_table.T)
    packed = _sc_gather_packed(table4, uid32)

    ncat = 32
    dtab = jnp.zeros((ncat, D), f32).at[: day_table.shape[0]].set(day_table)
    htab = jnp.zeros((ncat, D), f32).at[: hour_table.shape[0]].set(hour_table)

    w1u = W1[:, :D].T
    w1d = W1[:, D:2 * D].T
    w1h = W1[:, 2 * D:3 * D].T
    w1m = W1[:, 3 * D][None, :]
    b1r = b1[None, :]
    w2 = W2.T
    b2r = b2[None, :]
    n_out = 8
    w3 = jnp.zeros((W2.shape[0], n_out), f32).at[:, : W3.shape[0]].set(W3.T)
    b3r = jnp.zeros((1, n_out), f32).at[0, : W3.shape[0]].set(b3)

    uid2 = uid32[:, None]
    days2 = days.astype(jnp.int32)[:, None]
    hours2 = hours.astype(jnp.int32)[:, None]
    md2 = move_distance[:, None]

    out = _mlp_call(packed, uid2, days2, hours2, md2, dtab, htab, w1u, w1d,
                    w1h, w1m, b1r, w2, b2r, w3, b3r)
    return out[:, : W3.shape[0]]


# speedup vs baseline: 26.9040x; 1.4972x over previous
import functools

import jax
import jax.numpy as jnp
from jax import lax
from jax.experimental import pallas as pl
from jax.experimental.pallas import tpu as pltpu
from jax.experimental.pallas import tpu_sc as plsc

_RW = 16384
_W8 = _RW // 8
_SH_BLK = _RW.bit_length() - 1
_SH_S = _W8.bit_length() - 1
_MS = _W8 - 1


def _repack_body(x_ref, o_ref):
    bf16 = jnp.bfloat16
    eye = ((lax.broadcasted_iota(jnp.int32, (32, 32), 0)
            == lax.broadcasted_iota(jnp.int32, (32, 32), 1))
           .astype(bf16))
    xb = x_ref[...].astype(bf16)
    for k in range(4):
        lo = lax.dot_general(xb[:, k * _W8:(k + 1) * _W8], eye,
                             (((0,), (0,)), ((), ())),
                             preferred_element_type=jnp.float32).astype(bf16)
        hi = lax.dot_general(xb[:, (k + 4) * _W8:(k + 5) * _W8], eye,
                             (((0,), (0,)), ((), ())),
                             preferred_element_type=jnp.float32).astype(bf16)
        lo32 = lax.bitcast_convert_type(lo, jnp.uint16).astype(jnp.uint32)
        hi32 = lax.bitcast_convert_type(hi, jnp.uint16).astype(jnp.uint32)
        word = (hi32 << 16) | lo32
        o_ref[:, 32 * k:32 * (k + 1)] = lax.bitcast_convert_type(
            word, jnp.float32)


def _repack(tableT):
    V = tableT.shape[1]
    nb = (V + _RW - 1) // _RW
    return pl.pallas_call(
        _repack_body,
        grid=(nb,),
        in_specs=[pl.BlockSpec((32, _RW), lambda i: (0, i))],
        out_specs=pl.BlockSpec((_W8, 128), lambda i: (i, 0)),
        out_shape=jax.ShapeDtypeStruct((nb * _W8, 128), jnp.float32),
    )(tableT)


def _sc_gather_packed(table8, idx):
    B = idx.shape[0]
    D4 = table8.shape[1]
    info = plsc.get_sparse_core_info()
    NC, NS = info.num_cores, info.num_subcores
    L = info.num_lanes
    NW = NC * NS
    b_per_w = B // NW
    mesh = plsc.VectorSubcoreMesh(core_axis_name="c", subcore_axis_name="s")

    @functools.partial(
        pl.kernel,
        mesh=mesh,
        out_type=jax.ShapeDtypeStruct((B, D4), jnp.float32),
        scratch_types=[
            pltpu.VMEM((b_per_w,), jnp.int32),
            pltpu.VMEM((b_per_w,), jnp.int32),
            pltpu.VMEM((b_per_w, D4), jnp.float32),
            pltpu.SemaphoreType.DMA,
        ],
    )
    def gather_kernel(idx_hbm, table_hbm, out_hbm, idx_v, idx4_v, rows_v, sem):
        wid = lax.axis_index("s") * NC + lax.axis_index("c")
        base = wid * b_per_w
        pltpu.sync_copy(idx_hbm.at[pl.ds(base, b_per_w)], idx_v)
        for i in range(b_per_w // L):
            v = idx_v[pl.ds(i * L, L)]
            idx4_v[pl.ds(i * L, L)] = ((v >> _SH_BLK) << _SH_S) + (v & _MS)
        pltpu.async_copy(table_hbm.at[idx4_v], rows_v, sem).wait()
        pltpu.sync_copy(rows_v, out_hbm.at[pl.ds(base, b_per_w)])

    return gather_kernel(idx, table8)


_BT = 2048


def _mlp_body(p_ref, uid_ref, d_ref, h_ref, m_ref, dtab_ref, htab_ref,
              w1u_ref, w1d_ref, w1h_ref, w1m_ref, b1_ref, w2_ref, b2_ref,
              w3_ref, b3_ref, o_ref):
    f32 = jnp.float32
    bt = p_ref.shape[0]
    ncat = dtab_ref.shape[0]
    D = dtab_ref.shape[1]
    sub = (uid_ref[...] >> _SH_S) & 7
    k = sub & 3
    hi = sub >> 2
    words = lax.bitcast_convert_type(p_ref[...], jnp.uint32)
    grp = jnp.where(k == 0, words[:, :D], 0)
    for j in range(1, 4):
        grp = grp | jnp.where(k == j, words[:, j * D:(j + 1) * D], 0)
    lo_f = lax.bitcast_convert_type(grp << 16, f32)
    hi_f = lax.bitcast_convert_type(grp & jnp.uint32(0xFFFF0000), f32)
    uemb = jnp.where(hi == 1, hi_f, lo_f)
    doh = (d_ref[...] == lax.broadcasted_iota(jnp.int32, (bt, ncat), 1)).astype(f32)
    hoh = (h_ref[...] == lax.broadcasted_iota(jnp.int32, (bt, ncat), 1)).astype(f32)
    demb = jnp.dot(doh, dtab_ref[...], preferred_element_type=f32)
    hemb = jnp.dot(hoh, htab_ref[...], preferred_element_type=f32)
    acc = jnp.dot(uemb, w1u_ref[...], preferred_element_type=f32)
    acc = acc + jnp.dot(demb, w1d_ref[...], preferred_element_type=f32)
    acc = acc + jnp.dot(hemb, w1h_ref[...], preferred_element_type=f32)
    acc = acc + m_ref[...] * w1m_ref[...]
    h1 = jnp.maximum(acc + b1_ref[...], 0.0)
    h2 = jnp.maximum(
        jnp.dot(h1, w2_ref[...], preferred_element_type=f32) + b2_ref[...], 0.0)
    o_ref[...] = jnp.dot(h2, w3_ref[...], preferred_element_type=f32) + b3_ref[...]


def _mlp_call(packed, uid2, days2, hours2, md2, dtab, htab, w1u, w1d, w1h,
              w1m, b1r, w2, b2r, w3, b3r):
    B = packed.shape[0]
    n_out = w3.shape[1]
    bt = _BT
    grid = (B // bt,)

    def row_block(cols):
        return pl.BlockSpec((bt, cols), lambda i: (i, 0))

    def full(a):
        return pl.BlockSpec(a.shape, lambda i: (0,) * a.ndim)

    return pl.pallas_call(
        _mlp_body,
        grid=grid,
        in_specs=[
            row_block(packed.shape[1]),
            row_block(1), row_block(1), row_block(1), row_block(1),
            full(dtab), full(htab), full(w1u), full(w1d), full(w1h),
            full(w1m), full(b1r), full(w2), full(b2r), full(w3), full(b3r),
        ],
        out_specs=row_block(n_out),
        out_shape=jax.ShapeDtypeStruct((B, n_out), jnp.float32),
    )(packed, uid2, days2, hours2, md2, dtab, htab, w1u, w1d, w1h, w1m, b1r,
      w2, b2r, w3, b3r)


def kernel(user_ids, hours, days, move_distance, user_table, day_table,
           hour_table, W1, b1, W2, b2, W3, b3):
    B = user_ids.shape[0]
    D = user_table.shape[1]
    f32 = jnp.float32

    uid32 = user_ids.astype(jnp.int32)
    table8 = _repack(user_table.T)
    packed = _sc_gather_packed(table8, uid32)

    ncat = 32
    dtab = jnp.zeros((ncat, D), f32).at[: day_table.shape[0]].set(day_table)
    htab = jnp.zeros((ncat, D), f32).at[: hour_table.shape[0]].set(hour_table)

    w1u = W1[:, :D].T
    w1d = W1[:, D:2 * D].T
    w1h = W1[:, 2 * D:3 * D].T
    w1m = W1[:, 3 * D][None, :]
    b1r = b1[None, :]
    w2 = W2.T
    b2r = b2[None, :]
    n_out = 8
    w3 = jnp.zeros((W2.shape[0], n_out), f32).at[:, : W3.shape[0]].set(W3.T)
    b3r = jnp.zeros((1, n_out), f32).at[0, : W3.shape[0]].set(b3)

    uid2 = uid32[:, None]
    days2 = days.astype(jnp.int32)[:, None]
    hours2 = hours.astype(jnp.int32)[:, None]
    md2 = move_distance[:, None]

    out = _mlp_call(packed, uid2, days2, hours2, md2, dtab, htab, w1u, w1d,
                    w1h, w1m, b1r, w2, b2r, w3, b3r)
    return out[:, : W3.shape[0]]


# speedup vs baseline: 28.9147x; 1.0747x over previous
import functools

import jax
import jax.numpy as jnp
from jax import lax
from jax.experimental import pallas as pl
from jax.experimental.pallas import tpu as pltpu
from jax.experimental.pallas import tpu_sc as plsc

_RW = 16384
_W8 = _RW // 8
_SH_BLK = _RW.bit_length() - 1
_SH_S = _W8.bit_length() - 1
_MS = _W8 - 1


def _repack_body(x_ref, o_ref):
    bf16 = jnp.bfloat16
    eye = ((lax.broadcasted_iota(jnp.int32, (32, 32), 0)
            == lax.broadcasted_iota(jnp.int32, (32, 32), 1))
           .astype(bf16))
    xb = x_ref[...].astype(bf16)
    for k in range(4):
        lo = lax.dot_general(xb[:, k * _W8:(k + 1) * _W8], eye,
                             (((0,), (0,)), ((), ())),
                             preferred_element_type=jnp.float32).astype(bf16)
        hi = lax.dot_general(xb[:, (k + 4) * _W8:(k + 5) * _W8], eye,
                             (((0,), (0,)), ((), ())),
                             preferred_element_type=jnp.float32).astype(bf16)
        lo32 = lax.bitcast_convert_type(lo, jnp.uint16).astype(jnp.uint32)
        hi32 = lax.bitcast_convert_type(hi, jnp.uint16).astype(jnp.uint32)
        word = (hi32 << 16) | lo32
        o_ref[:, 32 * k:32 * (k + 1)] = lax.bitcast_convert_type(
            word, jnp.float32)


def _repack(tableT):
    V = tableT.shape[1]
    nb = (V + _RW - 1) // _RW
    return pl.pallas_call(
        _repack_body,
        grid=(nb,),
        in_specs=[pl.BlockSpec((32, _RW), lambda i: (0, i))],
        out_specs=pl.BlockSpec((_W8, 128), lambda i: (i, 0)),
        out_shape=jax.ShapeDtypeStruct((nb * _W8, 128), jnp.float32),
    )(tableT)


def _sc_gather_packed(table8, idx):
    B = idx.shape[0]
    D4 = table8.shape[1]
    info = plsc.get_sparse_core_info()
    NC, NS = info.num_cores, info.num_subcores
    L = info.num_lanes
    NW = NC * NS
    b_per_w = B // NW
    mesh = plsc.VectorSubcoreMesh(core_axis_name="c", subcore_axis_name="s")

    @functools.partial(
        pl.kernel,
        mesh=mesh,
        out_type=jax.ShapeDtypeStruct((B, D4), jnp.float32),
        scratch_types=[
            pltpu.VMEM((b_per_w,), jnp.int32),
            pltpu.VMEM((b_per_w,), jnp.int32),
            pltpu.VMEM((b_per_w, D4), jnp.float32),
            pltpu.SemaphoreType.DMA,
        ],
    )
    def gather_kernel(idx_hbm, table_hbm, out_hbm, idx_v, idx4_v, rows_v, sem):
        wid = lax.axis_index("s") * NC + lax.axis_index("c")
        base = wid * b_per_w
        pltpu.sync_copy(idx_hbm.at[pl.ds(base, b_per_w)], idx_v)
        for i in range(b_per_w // L):
            v = idx_v[pl.ds(i * L, L)]
            idx4_v[pl.ds(i * L, L)] = ((v >> _SH_BLK) << _SH_S) + (v & _MS)
        pltpu.async_copy(table_hbm.at[idx4_v], rows_v, sem).wait()
        pltpu.sync_copy(rows_v, out_hbm.at[pl.ds(base, b_per_w)])

    return gather_kernel(idx, table8)


_BT = 2048


def _mlp_body(p_ref, uid_ref, d_ref, h_ref, m_ref, dtab_ref, htab_ref,
              w1u_ref, w1d_ref, w1h_ref, w1m_ref, b1_ref, w2_ref, b2_ref,
              w3_ref, b3_ref, o_ref):
    f32 = jnp.float32
    bt = p_ref.shape[0]
    ncat = 32
    D = dtab_ref.shape[1]
    uid = uid_ref[...].T
    sub = (uid >> _SH_S) & 7
    k = sub & 3
    hi = sub >> 2
    words = lax.bitcast_convert_type(p_ref[...], jnp.uint32)
    grp = jnp.where(k == 0, words[:, :D], 0)
    for j in range(1, 4):
        grp = grp | jnp.where(k == j, words[:, j * D:(j + 1) * D], 0)
    lo_f = lax.bitcast_convert_type(grp << 16, f32)
    hi_f = lax.bitcast_convert_type(grp & jnp.uint32(0xFFFF0000), f32)
    uemb = jnp.where(hi == 1, hi_f, lo_f)
    dtab = jnp.concatenate(
        [dtab_ref[...],
         jnp.zeros((ncat - dtab_ref.shape[0], D), f32)], axis=0)
    htab = jnp.concatenate(
        [htab_ref[...],
         jnp.zeros((ncat - htab_ref.shape[0], D), f32)], axis=0)
    doh = (d_ref[...].T == lax.broadcasted_iota(jnp.int32, (bt, ncat), 1)).astype(f32)
    hoh = (h_ref[...].T == lax.broadcasted_iota(jnp.int32, (bt, ncat), 1)).astype(f32)
    demb = jnp.dot(doh, dtab, preferred_element_type=f32)
    hemb = jnp.dot(hoh, htab, preferred_element_type=f32)
    acc = jnp.dot(uemb, w1u_ref[...], preferred_element_type=f32)
    acc = acc + jnp.dot(demb, w1d_ref[...], preferred_element_type=f32)
    acc = acc + jnp.dot(hemb, w1h_ref[...], preferred_element_type=f32)
    acc = acc + m_ref[...].T * w1m_ref[...]
    h1 = jnp.maximum(acc + b1_ref[...], 0.0)
    h2 = jnp.maximum(
        jnp.dot(h1, w2_ref[...], preferred_element_type=f32) + b2_ref[...], 0.0)
    o_ref[...] = lax.dot_general(
        h2, w3_ref[...], (((1,), (1,)), ((), ())),
        preferred_element_type=f32) + b3_ref[...]


def _mlp_call(packed, uidr, daysr, hoursr, mdr, dtab, htab, w1u, w1d, w1h,
              w1m, b1r, w2, b2r, w3, b3r):
    B = packed.shape[0]
    n_out = w3.shape[0]
    bt = _BT
    grid = (B // bt,)

    def row_block(cols):
        return pl.BlockSpec((bt, cols), lambda i: (i, 0))

    def col_block():
        return pl.BlockSpec((1, bt), lambda i: (0, i))

    def full(a):
        return pl.BlockSpec(a.shape, lambda i: (0,) * a.ndim)

    return pl.pallas_call(
        _mlp_body,
        grid=grid,
        in_specs=[
            row_block(packed.shape[1]),
            col_block(), col_block(), col_block(), col_block(),
            full(dtab), full(htab), full(w1u), full(w1d), full(w1h),
            full(w1m), full(b1r), full(w2), full(b2r), full(w3), full(b3r),
        ],
        out_specs=row_block(n_out),
        out_shape=jax.ShapeDtypeStruct((B, n_out), jnp.float32),
    )(packed, uidr, daysr, hoursr, mdr, dtab, htab, w1u, w1d, w1h, w1m, b1r,
      w2, b2r, w3, b3r)


def kernel(user_ids, hours, days, move_distance, user_table, day_table,
           hour_table, W1, b1, W2, b2, W3, b3):
    B = user_ids.shape[0]
    D = user_table.shape[1]
    f32 = jnp.float32

    uid32 = user_ids.astype(jnp.int32)
    table8 = _repack(user_table.T)
    packed = _sc_gather_packed(table8, uid32)

    W1T = W1.T
    w1u = W1T[:D]
    w1d = W1T[D:2 * D]
    w1h = W1T[2 * D:3 * D]
    w1m = W1T[3 * D][None, :]
    b1r = b1[None, :]
    w2 = W2.T
    b2r = b2[None, :]
    b3r = b3[None, :]

    uidr = uid32[None, :]
    daysr = days.astype(jnp.int32)[None, :]
    hoursr = hours.astype(jnp.int32)[None, :]
    mdr = move_distance[None, :]

    return _mlp_call(packed, uidr, daysr, hoursr, mdr, day_table, hour_table,
                     w1u, w1d, w1h, w1m, b1r, w2, b2r, W3, b3r)


# speedup vs baseline: 28.9522x; 1.0013x over previous
import functools

import jax
import jax.numpy as jnp
from jax import lax
from jax.experimental import pallas as pl
from jax.experimental.pallas import tpu as pltpu
from jax.experimental.pallas import tpu_sc as plsc

_RW = 16384
_W8 = _RW // 8
_SH_BLK = _RW.bit_length() - 1
_SH_S = _W8.bit_length() - 1
_MS = _W8 - 1


def _repack_body(x_ref, o_ref):
    bf16 = jnp.bfloat16
    eye = ((lax.broadcasted_iota(jnp.int32, (32, 32), 0)
            == lax.broadcasted_iota(jnp.int32, (32, 32), 1))
           .astype(bf16))
    xb = x_ref[...].astype(bf16)
    for k in range(4):
        lo = lax.dot_general(xb[:, k * _W8:(k + 1) * _W8], eye,
                             (((0,), (0,)), ((), ())),
                             preferred_element_type=jnp.float32).astype(bf16)
        hi = lax.dot_general(xb[:, (k + 4) * _W8:(k + 5) * _W8], eye,
                             (((0,), (0,)), ((), ())),
                             preferred_element_type=jnp.float32).astype(bf16)
        lo32 = lax.bitcast_convert_type(lo, jnp.uint16).astype(jnp.uint32)
        hi32 = lax.bitcast_convert_type(hi, jnp.uint16).astype(jnp.uint32)
        word = (hi32 << 16) | lo32
        o_ref[:, 32 * k:32 * (k + 1)] = lax.bitcast_convert_type(
            word, jnp.float32)


def _repack(tableT):
    V = tableT.shape[1]
    nb = (V + _RW - 1) // _RW
    return pl.pallas_call(
        _repack_body,
        grid=(nb,),
        in_specs=[pl.BlockSpec((32, _RW), lambda i: (0, i))],
        out_specs=pl.BlockSpec((_W8, 128), lambda i: (i, 0)),
        out_shape=jax.ShapeDtypeStruct((nb * _W8, 128), jnp.float32),
    )(tableT)


def _sc_gather_packed(table8, idx):
    B = idx.shape[0]
    D4 = table8.shape[1]
    info = plsc.get_sparse_core_info()
    NC, NS = info.num_cores, info.num_subcores
    L = info.num_lanes
    NW = NC * NS
    b_per_w = B // NW
    mesh = plsc.VectorSubcoreMesh(core_axis_name="c", subcore_axis_name="s")

    @functools.partial(
        pl.kernel,
        mesh=mesh,
        out_type=jax.ShapeDtypeStruct((B, D4), jnp.float32),
        scratch_types=[
            pltpu.VMEM((b_per_w,), jnp.int32),
            pltpu.VMEM((b_per_w,), jnp.int32),
            pltpu.VMEM((b_per_w, D4), jnp.float32),
            pltpu.SemaphoreType.DMA,
        ],
    )
    def gather_kernel(idx_hbm, table_hbm, out_hbm, idx_v, idx4_v, rows_v, sem):
        wid = lax.axis_index("s") * NC + lax.axis_index("c")
        base = wid * b_per_w
        pltpu.sync_copy(idx_hbm.at[pl.ds(base, b_per_w)], idx_v)
        for i in range(b_per_w // L):
            v = idx_v[pl.ds(i * L, L)]
            idx4_v[pl.ds(i * L, L)] = ((v >> _SH_BLK) << _SH_S) + (v & _MS)
        pltpu.async_copy(table_hbm.at[idx4_v], rows_v, sem).wait()
        pltpu.sync_copy(rows_v, out_hbm.at[pl.ds(base, b_per_w)])

    return gather_kernel(idx, table8)


_BT = 2048


def _mlp_body(p_ref, uid_ref, d_ref, h_ref, m_ref, dtab_ref, htab_ref,
              w1_ref, b1_ref, w2_ref, b2_ref, w3_ref, b3_ref, o_ref):
    f32 = jnp.float32
    bt = p_ref.shape[0]
    ncat = 32
    D = dtab_ref.shape[1]
    w1u_ref = w1_ref.at[pl.ds(0, D)]
    w1d_ref = w1_ref.at[pl.ds(D, D)]
    w1h_ref = w1_ref.at[pl.ds(2 * D, D)]
    w1m_ref = w1_ref.at[pl.ds(3 * D, 1)]
    uid = uid_ref[...].T
    sub = (uid >> _SH_S) & 7
    k = sub & 3
    hi = sub >> 2
    words = lax.bitcast_convert_type(p_ref[...], jnp.uint32)
    grp = jnp.where(k == 0, words[:, :D], 0)
    for j in range(1, 4):
        grp = grp | jnp.where(k == j, words[:, j * D:(j + 1) * D], 0)
    lo_f = lax.bitcast_convert_type(grp << 16, f32)
    hi_f = lax.bitcast_convert_type(grp & jnp.uint32(0xFFFF0000), f32)
    uemb = jnp.where(hi == 1, hi_f, lo_f)
    dtab = jnp.concatenate(
        [dtab_ref[...],
         jnp.zeros((ncat - dtab_ref.shape[0], D), f32)], axis=0)
    htab = jnp.concatenate(
        [htab_ref[...],
         jnp.zeros((ncat - htab_ref.shape[0], D), f32)], axis=0)
    doh = (d_ref[...].T == lax.broadcasted_iota(jnp.int32, (bt, ncat), 1)).astype(f32)
    hoh = (h_ref[...].T == lax.broadcasted_iota(jnp.int32, (bt, ncat), 1)).astype(f32)
    demb = jnp.dot(doh, dtab, preferred_element_type=f32)
    hemb = jnp.dot(hoh, htab, preferred_element_type=f32)
    acc = jnp.dot(uemb, w1u_ref[...], preferred_element_type=f32)
    acc = acc + jnp.dot(demb, w1d_ref[...], preferred_element_type=f32)
    acc = acc + jnp.dot(hemb, w1h_ref[...], preferred_element_type=f32)
    acc = acc + m_ref[...].T * w1m_ref[...]
    h1 = jnp.maximum(acc + b1_ref[...], 0.0)
    h2 = jnp.maximum(
        lax.dot_general(h1, w2_ref[...], (((1,), (1,)), ((), ())),
                        preferred_element_type=f32) + b2_ref[...], 0.0)
    o_ref[...] = lax.dot_general(
        h2, w3_ref[...], (((1,), (1,)), ((), ())),
        preferred_element_type=f32) + b3_ref[...]


def _mlp_call(packed, uidr, daysr, hoursr, mdr, dtab, htab, w1t, b1r, w2,
              b2r, w3, b3r):
    B = packed.shape[0]
    n_out = w3.shape[0]
    bt = _BT
    grid = (B // bt,)

    def row_block(cols):
        return pl.BlockSpec((bt, cols), lambda i: (i, 0))

    def col_block():
        return pl.BlockSpec((1, bt), lambda i: (0, i))

    def full(a):
        return pl.BlockSpec(a.shape, lambda i: (0,) * a.ndim)

    return pl.pallas_call(
        _mlp_body,
        grid=grid,
        in_specs=[
            row_block(packed.shape[1]),
            col_block(), col_block(), col_block(), col_block(),
            full(dtab), full(htab), full(w1t), full(b1r), full(w2),
            full(b2r), full(w3), full(b3r),
        ],
        out_specs=row_block(n_out),
        out_shape=jax.ShapeDtypeStruct((B, n_out), jnp.float32),
    )(packed, uidr, daysr, hoursr, mdr, dtab, htab, w1t, b1r, w2, b2r, w3,
      b3r)


def kernel(user_ids, hours, days, move_distance, user_table, day_table,
           hour_table, W1, b1, W2, b2, W3, b3):
    B = user_ids.shape[0]
    D = user_table.shape[1]
    f32 = jnp.float32

    uid32 = user_ids.astype(jnp.int32)
    table8 = _repack(user_table.T)
    packed = _sc_gather_packed(table8, uid32)

    b1r = b1[None, :]
    b2r = b2[None, :]
    b3r = b3[None, :]

    uidr = uid32[None, :]
    daysr = days.astype(jnp.int32)[None, :]
    hoursr = hours.astype(jnp.int32)[None, :]
    mdr = move_distance[None, :]

    return _mlp_call(packed, uidr, daysr, hoursr, mdr, day_table, hour_table,
                     W1.T, b1r, W2, b2r, W3, b3r)


# speedup vs baseline: 29.8306x; 1.0303x over previous
import functools

import jax
import jax.numpy as jnp
from jax import lax
from jax.experimental import pallas as pl
from jax.experimental.pallas import tpu as pltpu
from jax.experimental.pallas import tpu_sc as plsc

_RW = 32768
_W8 = _RW // 8
_SH_BLK = _RW.bit_length() - 1
_SH_S = _W8.bit_length() - 1
_MS = _W8 - 1


def _repack_body(x_ref, o_ref):
    bf16 = jnp.bfloat16
    eye = ((lax.broadcasted_iota(jnp.int32, (32, 32), 0)
            == lax.broadcasted_iota(jnp.int32, (32, 32), 1))
           .astype(bf16))
    xb = x_ref[...].astype(bf16)
    for k in range(4):
        lo = lax.dot_general(xb[:, k * _W8:(k + 1) * _W8], eye,
                             (((0,), (0,)), ((), ())),
                             preferred_element_type=jnp.float32).astype(bf16)
        hi = lax.dot_general(xb[:, (k + 4) * _W8:(k + 5) * _W8], eye,
                             (((0,), (0,)), ((), ())),
                             preferred_element_type=jnp.float32).astype(bf16)
        lo32 = lax.bitcast_convert_type(lo, jnp.uint16).astype(jnp.uint32)
        hi32 = lax.bitcast_convert_type(hi, jnp.uint16).astype(jnp.uint32)
        word = (hi32 << 16) | lo32
        o_ref[:, 32 * k:32 * (k + 1)] = lax.bitcast_convert_type(
            word, jnp.float32)


def _repack(tableT):
    V = tableT.shape[1]
    nb = (V + _RW - 1) // _RW
    return pl.pallas_call(
        _repack_body,
        grid=(nb,),
        in_specs=[pl.BlockSpec((32, _RW), lambda i: (0, i))],
        out_specs=pl.BlockSpec((_W8, 128), lambda i: (i, 0)),
        out_shape=jax.ShapeDtypeStruct((nb * _W8, 128), jnp.float32),
    )(tableT)


def _sc_gather_packed(table8, idx):
    B = idx.shape[0]
    D4 = table8.shape[1]
    info = plsc.get_sparse_core_info()
    NC, NS = info.num_cores, info.num_subcores
    L = info.num_lanes
    NW = NC * NS
    b_per_w = B // NW
    mesh = plsc.VectorSubcoreMesh(core_axis_name="c", subcore_axis_name="s")

    @functools.partial(
        pl.kernel,
        mesh=mesh,
        out_type=jax.ShapeDtypeStruct((B, D4), jnp.float32),
        scratch_types=[
            pltpu.VMEM((b_per_w,), jnp.int32),
            pltpu.VMEM((b_per_w,), jnp.int32),
            pltpu.VMEM((b_per_w, D4), jnp.float32),
            pltpu.SemaphoreType.DMA,
        ],
    )
    def gather_kernel(idx_hbm, table_hbm, out_hbm, idx_v, idx4_v, rows_v, sem):
        wid = lax.axis_index("s") * NC + lax.axis_index("c")
        base = wid * b_per_w
        pltpu.sync_copy(idx_hbm.at[pl.ds(base, b_per_w)], idx_v)
        for i in range(b_per_w // L):
            v = idx_v[pl.ds(i * L, L)]
            idx4_v[pl.ds(i * L, L)] = ((v >> _SH_BLK) << _SH_S) + (v & _MS)
        pltpu.async_copy(table_hbm.at[idx4_v], rows_v, sem).wait()
        pltpu.sync_copy(rows_v, out_hbm.at[pl.ds(base, b_per_w)])

    return gather_kernel(idx, table8)


_BT = 4096


def _mlp_body(p_ref, uid_ref, d_ref, h_ref, m_ref, dtab_ref, htab_ref,
              w1_ref, b1_ref, w2_ref, b2_ref, w3_ref, b3_ref, o_ref):
    f32 = jnp.float32
    bt = p_ref.shape[0]
    ncat = 32
    D = dtab_ref.shape[1]
    w1u_ref = w1_ref.at[pl.ds(0, D)]
    w1d_ref = w1_ref.at[pl.ds(D, D)]
    w1h_ref = w1_ref.at[pl.ds(2 * D, D)]
    w1m_ref = w1_ref.at[pl.ds(3 * D, 1)]
    uid = uid_ref[...].T
    sub = (uid >> _SH_S) & 7
    k = sub & 3
    hi = sub >> 2
    words = lax.bitcast_convert_type(p_ref[...], jnp.uint32)
    grp = jnp.where(k == 0, words[:, :D], 0)
    for j in range(1, 4):
        grp = grp | jnp.where(k == j, words[:, j * D:(j + 1) * D], 0)
    lo_f = lax.bitcast_convert_type(grp << 16, f32)
    hi_f = lax.bitcast_convert_type(grp & jnp.uint32(0xFFFF0000), f32)
    uemb = jnp.where(hi == 1, hi_f, lo_f)
    dtab = jnp.concatenate(
        [dtab_ref[...],
         jnp.zeros((ncat - dtab_ref.shape[0], D), f32)], axis=0)
    htab = jnp.concatenate(
        [htab_ref[...],
         jnp.zeros((ncat - htab_ref.shape[0], D), f32)], axis=0)
    doh = (d_ref[...].T == lax.broadcasted_iota(jnp.int32, (bt, ncat), 1)).astype(f32)
    hoh = (h_ref[...].T == lax.broadcasted_iota(jnp.int32, (bt, ncat), 1)).astype(f32)
    demb = jnp.dot(doh, dtab, preferred_element_type=f32)
    hemb = jnp.dot(hoh, htab, preferred_element_type=f32)
    acc = jnp.dot(uemb, w1u_ref[...], preferred_element_type=f32)
    acc = acc + jnp.dot(demb, w1d_ref[...], preferred_element_type=f32)
    acc = acc + jnp.dot(hemb, w1h_ref[...], preferred_element_type=f32)
    acc = acc + m_ref[...].T * w1m_ref[...]
    h1 = jnp.maximum(acc + b1_ref[...], 0.0)
    h2 = jnp.maximum(
        lax.dot_general(h1, w2_ref[...], (((1,), (1,)), ((), ())),
                        preferred_element_type=f32) + b2_ref[...], 0.0)
    o_ref[...] = lax.dot_general(
        h2, w3_ref[...], (((1,), (1,)), ((), ())),
        preferred_element_type=f32) + b3_ref[...]


def _mlp_call(packed, uidr, daysr, hoursr, mdr, dtab, htab, w1t, b1r, w2,
              b2r, w3, b3r):
    B = packed.shape[0]
    n_out = w3.shape[0]
    bt = _BT
    grid = (B // bt,)

    def row_block(cols):
        return pl.BlockSpec((bt, cols), lambda i: (i, 0))

    def col_block():
        return pl.BlockSpec((1, bt), lambda i: (0, i))

    def full(a):
        return pl.BlockSpec(a.shape, lambda i: (0,) * a.ndim)

    return pl.pallas_call(
        _mlp_body,
        grid=grid,
        in_specs=[
            row_block(packed.shape[1]),
            col_block(), col_block(), col_block(), col_block(),
            full(dtab), full(htab), full(w1t), full(b1r), full(w2),
            full(b2r), full(w3), full(b3r),
        ],
        out_specs=row_block(n_out),
        out_shape=jax.ShapeDtypeStruct((B, n_out), jnp.float32),
    )(packed, uidr, daysr, hoursr, mdr, dtab, htab, w1t, b1r, w2, b2r, w3,
      b3r)


def kernel(user_ids, hours, days, move_distance, user_table, day_table,
           hour_table, W1, b1, W2, b2, W3, b3):
    B = user_ids.shape[0]
    D = user_table.shape[1]
    f32 = jnp.float32

    uid32 = user_ids.astype(jnp.int32)
    table8 = _repack(user_table.T)
    packed = _sc_gather_packed(table8, uid32)

    b1r = b1[None, :]
    b2r = b2[None, :]
    b3r = b3[None, :]

    uidr = uid32[None, :]
    daysr = days.astype(jnp.int32)[None, :]
    hoursr = hours.astype(jnp.int32)[None, :]
    mdr = move_distance[None, :]

    return _mlp_call(packed, uidr, daysr, hoursr, mdr, day_table, hour_table,
                     W1.T, b1r, W2, b2r, W3, b3r)
